# baseline (speedup 1.0000x reference)
import functools

import jax
import jax.numpy as jnp
from jax import lax
from jax.experimental import pallas as pl
from jax.experimental.pallas import tpu as pltpu
from jax.experimental.pallas import tpu_sc as plsc

IN_FEATURES = 16
OUT_FEATURES = 16
NUM_KNOTS = 10
NUM_SEG = NUM_KNOTS - 1
DEGREE = 3
BATCH = 16384

NUM_CORES = 2
NUM_SUBCORES = 16
LANES = 16
NUM_WORKERS = NUM_CORES * NUM_SUBCORES
ROWS_PER_W = BATCH // NUM_WORKERS
TABLE_SIZE = IN_FEATURES * NUM_SEG * (DEGREE + 1) * OUT_FEATURES

ROWS_PER_BLOCK = 128 // IN_FEATURES
NUM_BLOCKS = BATCH // ROWS_PER_BLOCK
BLOCKS_PER_W = NUM_BLOCKS // NUM_WORKERS


_GATHER_DNUMS = lax.GatherDimensionNumbers(
    offset_dims=(), collapsed_slice_dims=(0,), start_index_map=(0,))


def _lane_bcast(v, i):
    idx = jnp.full((LANES, 1), i, jnp.int32)
    return lax.gather(v, idx, _GATHER_DNUMS, slice_sizes=(1,),
                      mode=lax.GatherScatterMode.PROMISE_IN_BOUNDS)


def _kan_body(x_hbm, tab_hbm, kn_hbm, sc_hbm, out_hbm, xo_v, tab_v, kn_v,
              sc_v):
    cid = lax.axis_index("c")
    sid = lax.axis_index("s")
    wid = sid * NUM_CORES + cid
    row0 = wid * ROWS_PER_W
    pltpu.sync_copy(x_hbm.at[pl.ds(row0, ROWS_PER_W), :], xo_v)
    pltpu.sync_copy(tab_hbm, tab_v)
    pltpu.sync_copy(kn_hbm, kn_v)
    pltpu.sync_copy(sc_hbm, sc_v)

    iota = lax.iota(jnp.int32, LANES)
    scale_vec = sc_v[...]
    kns = [kn_v[j] for j in range(NUM_SEG - 1)]
    iotad = [iota + LANES * d for d in range(DEGREE + 1)]
    addr_base = iota * (NUM_SEG * (DEGREE + 1) * LANES)

    @plsc.parallel_loop(0, ROWS_PER_W, step=1, unroll=8)
    def row(r):
        xc = jnp.clip(xo_v[r], -1.0, 1.0)
        seg = jnp.zeros((LANES,), jnp.int32)
        for kb in kns:
            seg = seg + (kb < xc).astype(jnp.int32)
        y = (xc + 1.0) * (NUM_SEG / 2.0)
        t = y - seg.astype(jnp.float32)
        addr = addr_base + seg * ((DEGREE + 1) * LANES)
        acc = jnp.zeros((LANES,), jnp.float32)
        for i in range(IN_FEATURES):
            tb = _lane_bcast(t, i)
            ab = _lane_bcast(addr, i)
            c0 = plsc.load_gather(tab_v, [ab + iotad[0]])
            c1 = plsc.load_gather(tab_v, [ab + iotad[1]])
            c2 = plsc.load_gather(tab_v, [ab + iotad[2]])
            c3 = plsc.load_gather(tab_v, [ab + iotad[3]])
            acc = acc + (((c3 * tb + c2) * tb + c1) * tb + c0)
        xo_v[r] = acc * scale_vec

    pltpu.sync_copy(xo_v, out_hbm.at[pl.ds(row0, ROWS_PER_W), :])


@jax.jit
def kernel(x, coeffs, scale, knots):
    table = jnp.transpose(coeffs, (1, 2, 3, 0)).reshape(-1)
    kn_rows = jnp.broadcast_to(knots[1:NUM_SEG, None], (NUM_SEG - 1, LANES))
    mesh = plsc.VectorSubcoreMesh(core_axis_name="c", subcore_axis_name="s")
    run = pl.kernel(
        _kan_body,
        out_type=jax.ShapeDtypeStruct((BATCH, OUT_FEATURES), jnp.float32),
        mesh=mesh,
        compiler_params=pltpu.CompilerParams(needs_layout_passes=False),
        scratch_types=[
            pltpu.VMEM((ROWS_PER_W, IN_FEATURES), jnp.float32),
            pltpu.VMEM((TABLE_SIZE,), jnp.float32),
            pltpu.VMEM((NUM_SEG - 1, LANES), jnp.float32),
            pltpu.VMEM((LANES,), jnp.float32),
        ],
    )
    return run(x, table, kn_rows, scale)


# speedup vs baseline: 1.0679x; 1.0679x over previous
import functools

import jax
import jax.numpy as jnp
from jax import lax
from jax.experimental import pallas as pl
from jax.experimental.pallas import tpu as pltpu
from jax.experimental.pallas import tpu_sc as plsc

IN_FEATURES = 16
OUT_FEATURES = 16
NUM_KNOTS = 10
NUM_SEG = NUM_KNOTS - 1
DEGREE = 3
BATCH = 16384

NUM_CORES = 2
NUM_SUBCORES = 16
LANES = 16
NUM_WORKERS = NUM_CORES * NUM_SUBCORES
ROWS_PER_W = BATCH // NUM_WORKERS
TABLE_SIZE = IN_FEATURES * NUM_SEG * (DEGREE + 1) * OUT_FEATURES

ROWS_PER_BLOCK = 128 // IN_FEATURES
NUM_BLOCKS = BATCH // ROWS_PER_BLOCK
BLOCKS_PER_W = NUM_BLOCKS // NUM_WORKERS


_GATHER_DNUMS = lax.GatherDimensionNumbers(
    offset_dims=(), collapsed_slice_dims=(0,), start_index_map=(0,))


def _lane_bcast(v, i):
    idx = jnp.full((LANES, 1), i, jnp.int32)
    return lax.gather(v, idx, _GATHER_DNUMS, slice_sizes=(1,),
                      mode=lax.GatherScatterMode.PROMISE_IN_BOUNDS)


def _kan_body(x_hbm, tab_hbm, sc_hbm, out_hbm, xo_v, tab_v, sc_v):
    cid = lax.axis_index("c")
    sid = lax.axis_index("s")
    wid = sid * NUM_CORES + cid
    row0 = wid * ROWS_PER_W
    pltpu.sync_copy(x_hbm.at[pl.ds(row0, ROWS_PER_W), :], xo_v)
    pltpu.sync_copy(tab_hbm, tab_v)
    pltpu.sync_copy(sc_hbm, sc_v)

    iota = lax.iota(jnp.int32, LANES)
    scale_vec = sc_v[...]
    iotad = [iota + LANES * d for d in range(DEGREE + 1)]
    addr_base = iota * (NUM_SEG * (DEGREE + 1) * LANES)

    @plsc.parallel_loop(0, ROWS_PER_W, step=1, unroll=8)
    def row(r):
        xc = jnp.clip(xo_v[r], -1.0, 1.0)
        y = (xc + 1.0) * (NUM_SEG / 2.0)
        seg = jnp.minimum(y.astype(jnp.int32), NUM_SEG - 1)
        t = y - seg.astype(jnp.float32)
        addr = addr_base + seg * ((DEGREE + 1) * LANES)
        acc = jnp.zeros((LANES,), jnp.float32)
        for i in range(IN_FEATURES):
            tb = _lane_bcast(t, i)
            ab = _lane_bcast(addr, i)
            c0 = plsc.load_gather(tab_v, [ab + iotad[0]])
            c1 = plsc.load_gather(tab_v, [ab + iotad[1]])
            c2 = plsc.load_gather(tab_v, [ab + iotad[2]])
            c3 = plsc.load_gather(tab_v, [ab + iotad[3]])
            acc = acc + (((c3 * tb + c2) * tb + c1) * tb + c0)
        xo_v[r] = acc * scale_vec

    pltpu.sync_copy(xo_v, out_hbm.at[pl.ds(row0, ROWS_PER_W), :])


@jax.jit
def kernel(x, coeffs, scale, knots):
    table = jnp.transpose(coeffs, (1, 2, 3, 0)).reshape(-1)
    mesh = plsc.VectorSubcoreMesh(core_axis_name="c", subcore_axis_name="s")
    run = pl.kernel(
        _kan_body,
        out_type=jax.ShapeDtypeStruct((BATCH, OUT_FEATURES), jnp.float32),
        mesh=mesh,
        compiler_params=pltpu.CompilerParams(needs_layout_passes=False),
        scratch_types=[
            pltpu.VMEM((ROWS_PER_W, IN_FEATURES), jnp.float32),
            pltpu.VMEM((TABLE_SIZE,), jnp.float32),
            pltpu.VMEM((LANES,), jnp.float32),
        ],
    )
    return run(x, table, scale)


# speedup vs baseline: 1.0684x; 1.0005x over previous
import jax
import jax.numpy as jnp
from jax import lax
from jax.experimental import pallas as pl
from jax.experimental.pallas import tpu as pltpu
from jax.experimental.pallas import tpu_sc as plsc

IN_FEATURES = 16
OUT_FEATURES = 16
NUM_KNOTS = 10
NUM_SEG = NUM_KNOTS - 1
DEGREE = 3
BATCH = 16384

NUM_CORES = 2
NUM_SUBCORES = 16
LANES = 16
NUM_WORKERS = NUM_CORES * NUM_SUBCORES
ROWS_PER_W = BATCH // NUM_WORKERS
TABLE_SIZE = IN_FEATURES * NUM_SEG * (DEGREE + 1) * OUT_FEATURES


_GATHER_DNUMS = lax.GatherDimensionNumbers(
    offset_dims=(), collapsed_slice_dims=(0,), start_index_map=(0,))


def _lane_bcast(v, i):
    idx = jnp.full((LANES, 1), i, jnp.int32)
    return lax.gather(v, idx, _GATHER_DNUMS, slice_sizes=(1,),
                      mode=lax.GatherScatterMode.PROMISE_IN_BOUNDS)


def _kan_body(x_hbm, tab_hbm, sc_hbm, out_hbm, xo_v, tab_v, sc_v):
    cid = lax.axis_index("c")
    sid = lax.axis_index("s")
    wid = sid * NUM_CORES + cid
    row0 = wid * ROWS_PER_W
    pltpu.sync_copy(x_hbm.at[pl.ds(row0, ROWS_PER_W), :], xo_v)
    pltpu.sync_copy(tab_hbm, tab_v)
    pltpu.sync_copy(sc_hbm, sc_v)

    iota = lax.iota(jnp.int32, LANES)
    scale_vec = sc_v[...]
    iotad = [iota + LANES * d for d in range(DEGREE + 1)]
    addr_base = iota * (NUM_SEG * (DEGREE + 1) * LANES)

    @plsc.parallel_loop(0, ROWS_PER_W, step=1, unroll=8)
    def row(r):
        xc = jnp.clip(xo_v[r], -1.0, 1.0)
        y = (xc + 1.0) * (NUM_SEG / 2.0)
        seg = jnp.minimum(y.astype(jnp.int32), NUM_SEG - 1)
        t = y - seg.astype(jnp.float32)
        addr = addr_base + seg * ((DEGREE + 1) * LANES)
        acc = jnp.zeros((LANES,), jnp.float32)
        for i in range(IN_FEATURES):
            tb = _lane_bcast(t, i)
            ab = _lane_bcast(addr, i)
            c0 = plsc.load_gather(tab_v, [ab + iotad[0]])
            c1 = plsc.load_gather(tab_v, [ab + iotad[1]])
            c2 = plsc.load_gather(tab_v, [ab + iotad[2]])
            c3 = plsc.load_gather(tab_v, [ab + iotad[3]])
            acc = acc + (((c3 * tb + c2) * tb + c1) * tb + c0)
        xo_v[r] = acc * scale_vec

    pltpu.sync_copy(xo_v, out_hbm.at[pl.ds(row0, ROWS_PER_W), :])


@jax.jit
def kernel(x, coeffs, scale, knots):
    table = jnp.transpose(coeffs, (1, 2, 3, 0)).reshape(-1)
    mesh = plsc.VectorSubcoreMesh(core_axis_name="c", subcore_axis_name="s")
    run = pl.kernel(
        _kan_body,
        out_type=jax.ShapeDtypeStruct((BATCH, OUT_FEATURES), jnp.float32),
        mesh=mesh,
        compiler_params=pltpu.CompilerParams(needs_layout_passes=False),
        scratch_types=[
            pltpu.VMEM((ROWS_PER_W, IN_FEATURES), jnp.float32),
            pltpu.VMEM((TABLE_SIZE,), jnp.float32),
            pltpu.VMEM((LANES,), jnp.float32),
        ],
    )
    return run(x, table, scale)


# speedup vs baseline: 1.0888x; 1.0190x over previous
import jax
import jax.numpy as jnp
from jax import lax
from jax.experimental import pallas as pl
from jax.experimental.pallas import tpu as pltpu
from jax.experimental.pallas import tpu_sc as plsc

IN_FEATURES = 16
OUT_FEATURES = 16
NUM_KNOTS = 10
NUM_SEG = NUM_KNOTS - 1
DEGREE = 3
BATCH = 16384

NUM_CORES = 2
NUM_SUBCORES = 16
LANES = 16
NUM_WORKERS = NUM_CORES * NUM_SUBCORES
ROWS_PER_W = BATCH // NUM_WORKERS
TABLE_SIZE = IN_FEATURES * NUM_SEG * (DEGREE + 1) * OUT_FEATURES


_GATHER_DNUMS = lax.GatherDimensionNumbers(
    offset_dims=(), collapsed_slice_dims=(0,), start_index_map=(0,))


def _lane_bcast(v, i):
    idx = jnp.full((LANES, 1), i, jnp.int32)
    return lax.gather(v, idx, _GATHER_DNUMS, slice_sizes=(1,),
                      mode=lax.GatherScatterMode.PROMISE_IN_BOUNDS)


def _kan_body(x_hbm, tab_hbm, sc_hbm, out_hbm, xo_v, tab_v, sc_v, sem):
    cid = lax.axis_index("c")
    sid = lax.axis_index("s")
    wid = sid * NUM_CORES + cid
    row0 = wid * ROWS_PER_W
    cp1 = pltpu.async_copy(x_hbm.at[pl.ds(row0, ROWS_PER_W), :], xo_v, sem)
    cp2 = pltpu.async_copy(tab_hbm, tab_v, sem)
    cp3 = pltpu.async_copy(sc_hbm, sc_v, sem)
    cp1.wait()
    cp2.wait()
    cp3.wait()

    iota = lax.iota(jnp.int32, LANES)
    scale_vec = sc_v[...]
    iotad = [iota + LANES * d for d in range(DEGREE + 1)]
    addr_base = iota * (NUM_SEG * (DEGREE + 1) * LANES)

    @plsc.parallel_loop(0, ROWS_PER_W, step=1, unroll=8)
    def row(r):
        xc = jnp.clip(xo_v[r], -1.0, 1.0)
        y = (xc + 1.0) * (NUM_SEG / 2.0)
        seg = jnp.minimum(y.astype(jnp.int32), NUM_SEG - 1)
        t = y - seg.astype(jnp.float32)
        addr = addr_base + seg * ((DEGREE + 1) * LANES)
        acc = jnp.zeros((LANES,), jnp.float32)
        for i in range(IN_FEATURES):
            tb = _lane_bcast(t, i)
            ab = _lane_bcast(addr, i)
            c0 = plsc.load_gather(tab_v, [ab + iotad[0]])
            c1 = plsc.load_gather(tab_v, [ab + iotad[1]])
            c2 = plsc.load_gather(tab_v, [ab + iotad[2]])
            c3 = plsc.load_gather(tab_v, [ab + iotad[3]])
            acc = acc + (((c3 * tb + c2) * tb + c1) * tb + c0)
        xo_v[r] = acc * scale_vec

    pltpu.sync_copy(xo_v, out_hbm.at[pl.ds(row0, ROWS_PER_W), :])


@jax.jit
def kernel(x, coeffs, scale, knots):
    table = jnp.transpose(coeffs, (1, 2, 3, 0)).reshape(-1)
    mesh = plsc.VectorSubcoreMesh(core_axis_name="c", subcore_axis_name="s")
    run = pl.kernel(
        _kan_body,
        out_type=jax.ShapeDtypeStruct((BATCH, OUT_FEATURES), jnp.float32),
        mesh=mesh,
        compiler_params=pltpu.CompilerParams(needs_layout_passes=False),
        scratch_types=[
            pltpu.VMEM((ROWS_PER_W, IN_FEATURES), jnp.float32),
            pltpu.VMEM((TABLE_SIZE,), jnp.float32),
            pltpu.VMEM((LANES,), jnp.float32),
            pltpu.SemaphoreType.DMA,
        ],
    )
    return run(x, table, scale)
